# baseline (speedup 1.0000x reference)
import jax
import jax.numpy as jnp
from jax import lax
from jax.experimental import pallas as pl
from jax.experimental.pallas import tpu as pltpu
from jax.experimental.pallas import tpu_sc as plsc

N = 10000
E = 320000
D = 128

NC = 2
NS = 16
NW = NC * NS

CHUNK = 128
CPT = 80
EPT = CPT * CHUNK
E_PAD = NW * EPT

RPT = 632
NPAD = RPT * NS

_mesh = plsc.VectorSubcoreMesh(
    core_axis_name="c", subcore_axis_name="s", num_cores=NC, num_subcores=NS
)


HR = NPAD // D


def _deg_body(dst2, out, dst_v, hist_v):
    c = lax.axis_index("c")
    s = lax.axis_index("s")
    wid = c * NS + s

    def zbody(r, carry):
        for j in range(8):
            hist_v[r, pl.ds(j * 16, 16)] = jnp.zeros((16,), jnp.float32)
        return carry

    lax.fori_loop(0, HR, zbody, 0)
    pltpu.sync_copy(dst2.at[pl.ds(wid * CPT, CPT)], dst_v)

    ones = jnp.ones((16,), jnp.float32)

    def body(r, carry):
        for j in range(CHUNK // 16):
            idx = dst_v[r, pl.ds(j * 16, 16)]
            row = lax.shift_right_logical(idx, 7)
            col = lax.bitwise_and(idx, 127)
            plsc.addupdate_scatter(hist_v, [row, col], ones)
        return carry

    lax.fori_loop(0, CPT, body, 0)
    pltpu.sync_copy(hist_v, out.at[wid])


_deg_call = pl.kernel(
    _deg_body,
    out_type=jax.ShapeDtypeStruct((NW, HR, D), jnp.float32),
    mesh=_mesh,
    scratch_types=[
        pltpu.VMEM((CPT, CHUNK), jnp.int32),
        pltpu.VMEM((HR, D), jnp.float32),
    ],
    compiler_params=pltpu.CompilerParams(needs_layout_passes=False),
)


def _edge_body(h, packed, zeros, out, ia, ib, rows_a, rows_b,
               sem_ga, sem_gb, sem_ia, sem_ib, acc):
    c = lax.axis_index("c")
    s = lax.axis_index("s")
    wid = c * NS + s
    r0 = s * RPT
    base = wid * CPT
    pltpu.sync_copy(zeros.at[pl.ds(r0, RPT)], acc.at[pl.ds(r0, RPT)])
    pltpu.sync_copy(packed.at[pl.ds(base, 1)], ia)
    pltpu.sync_copy(packed.at[pl.ds(base + 1, 1)], ib)
    plsc.subcore_barrier()

    pltpu.async_copy(h.at[ia.at[0, 0]], rows_a, sem_ga)

    def body(i, carry):
        k = 2 * i
        pltpu.async_copy(h.at[ib.at[0, 0]], rows_b, sem_gb)
        pltpu.make_async_copy(h.at[ia.at[0, 0]], rows_a, sem_ga).wait()
        pltpu.sync_copy(rows_a, acc.at[ia.at[0, 1]], add=True)
        pltpu.async_copy(packed.at[pl.ds(base + k + 2, 1)], ia, sem_ia).wait()
        pltpu.async_copy(h.at[ia.at[0, 0]], rows_a, sem_ga)
        pltpu.make_async_copy(h.at[ib.at[0, 0]], rows_b, sem_gb).wait()
        pltpu.sync_copy(rows_b, acc.at[ib.at[0, 1]], add=True)
        pltpu.async_copy(packed.at[pl.ds(base + k + 3, 1)], ib, sem_ib).wait()
        return carry

    lax.fori_loop(0, CPT // 2 - 1, body, 0)
    pltpu.async_copy(h.at[ib.at[0, 0]], rows_b, sem_gb)
    pltpu.make_async_copy(h.at[ia.at[0, 0]], rows_a, sem_ga).wait()
    pltpu.sync_copy(rows_a, acc.at[ia.at[0, 1]], add=True)
    pltpu.make_async_copy(h.at[ib.at[0, 0]], rows_b, sem_gb).wait()
    pltpu.sync_copy(rows_b, acc.at[ib.at[0, 1]], add=True)

    plsc.subcore_barrier()
    pltpu.sync_copy(acc.at[pl.ds(r0, RPT)], out.at[c, pl.ds(r0, RPT)])


_edge_call = pl.kernel(
    _edge_body,
    out_type=jax.ShapeDtypeStruct((NC, NPAD, D), jnp.float32),
    mesh=_mesh,
    scratch_types=[
        pltpu.VMEM((1, 2, CHUNK), jnp.int32),
        pltpu.VMEM((1, 2, CHUNK), jnp.int32),
        pltpu.VMEM((CHUNK, D), jnp.float32),
        pltpu.VMEM((CHUNK, D), jnp.float32),
        pltpu.SemaphoreType.DMA,
        pltpu.SemaphoreType.DMA,
        pltpu.SemaphoreType.DMA,
        pltpu.SemaphoreType.DMA,
        pltpu.VMEM_SHARED((NPAD, D), jnp.float32),
    ],
)


def _tc0_body(degp_ref, dis2_ref):
    dsum = degp_ref[0]
    for i in range(1, NW):
        dsum = dsum + degp_ref[i]
    dis2_ref[...] = lax.rsqrt(dsum + 1.0)


_tc0_call = pl.pallas_call(
    _tc0_body,
    out_shape=jax.ShapeDtypeStruct((HR, D), jnp.float32),
)


def _tc1_body(x_ref, w_ref, dis_ref, hp_ref):
    h = jnp.dot(x_ref[...], w_ref[...], preferred_element_type=jnp.float32)
    hp_ref[...] = h * dis_ref[...]


_tc1_call = pl.pallas_call(
    _tc1_body,
    out_shape=jax.ShapeDtypeStruct((N, D), jnp.float32),
)


def _tc2_body(s_ref, hp_ref, dis_ref, b_ref, w_ref, out_ref):
    dis = dis_ref[...]
    t = (s_ref[0, :N] + s_ref[1, :N] + hp_ref[...]) * dis + b_ref[...]
    h1 = jnp.maximum(t, 0.0)
    out_ref[...] = jnp.dot(h1, w_ref[...], preferred_element_type=jnp.float32) * dis


_tc2_call = pl.pallas_call(
    _tc2_body,
    out_shape=jax.ShapeDtypeStruct((N, D), jnp.float32),
)


def _tc3_body(s_ref, hp_ref, dis_ref, b_ref, out_ref):
    t = (s_ref[0, :N] + s_ref[1, :N] + hp_ref[...]) * dis_ref[...] + b_ref[...]
    out_ref[...] = 1.0 / (1.0 + jnp.exp(-t))


_tc3_call = pl.pallas_call(
    _tc3_body,
    out_shape=jax.ShapeDtypeStruct((N, D), jnp.float32),
)


@jax.jit
def kernel(x, edge_index, W0, b0, W1, b1):
    src = edge_index[0]
    dst = edge_index[1]
    pad = E_PAD - E
    iota_pad = jnp.arange(pad, dtype=jnp.int32)
    src_pad = (iota_pad * 131) % N
    dst_pad = N + iota_pad % (NPAD - N)
    src2 = jnp.concatenate([src, src_pad]).reshape(-1, CHUNK)
    dst2 = jnp.concatenate([dst, dst_pad]).reshape(-1, CHUNK)
    packed = jnp.stack([src2, dst2], axis=1)

    zeros = jnp.zeros((NPAD, D), jnp.float32)

    degp = _deg_call(dst2)
    dis = _tc0_call(degp).reshape(NPAD, 1)[:N]
    h0p = _tc1_call(x, W0, dis)

    s1 = _edge_call(h0p, packed, zeros)
    h1p = _tc2_call(s1, h0p, dis, b0.reshape(1, D), W1)

    s2 = _edge_call(h1p, packed, zeros)
    return _tc3_call(s2, h1p, dis, b1.reshape(1, D))


# speedup vs baseline: 1.1092x; 1.1092x over previous
import jax
import jax.numpy as jnp
from jax import lax
from jax.experimental import pallas as pl
from jax.experimental.pallas import tpu as pltpu
from jax.experimental.pallas import tpu_sc as plsc

N = 10000
E = 320000
D = 128

NC = 2
NS = 16
NW = NC * NS

CHUNK = 128
CPT = 80
EPT = CPT * CHUNK
E_PAD = NW * EPT

RPT = 632
NPAD = RPT * NS

_mesh = plsc.VectorSubcoreMesh(
    core_axis_name="c", subcore_axis_name="s", num_cores=NC, num_subcores=NS
)


HR = NPAD // D


def _deg_body(dst2, out, dst_v, hist_v):
    c = lax.axis_index("c")
    s = lax.axis_index("s")
    wid = c * NS + s

    def zbody(r, carry):
        for j in range(8):
            hist_v[r, pl.ds(j * 16, 16)] = jnp.zeros((16,), jnp.float32)
        return carry

    lax.fori_loop(0, HR, zbody, 0)
    pltpu.sync_copy(dst2.at[pl.ds(wid * CPT, CPT)], dst_v)

    ones = jnp.ones((16,), jnp.float32)

    def body(r, carry):
        for j in range(CHUNK // 16):
            idx = dst_v[r, pl.ds(j * 16, 16)]
            row = lax.shift_right_logical(idx, 7)
            col = lax.bitwise_and(idx, 127)
            plsc.addupdate_scatter(hist_v, [row, col], ones)
        return carry

    lax.fori_loop(0, CPT, body, 0)
    pltpu.sync_copy(hist_v, out.at[wid])


_deg_call = pl.kernel(
    _deg_body,
    out_type=jax.ShapeDtypeStruct((NW, HR, D), jnp.float32),
    mesh=_mesh,
    scratch_types=[
        pltpu.VMEM((CPT, CHUNK), jnp.int32),
        pltpu.VMEM((HR, D), jnp.float32),
    ],
    compiler_params=pltpu.CompilerParams(needs_layout_passes=False),
)


def _edge_body(h, packed, zeros, out, ia, ib, rows_a, rows_b,
               sem_ga, sem_gb, sem_ia, sem_ib, ia2, ib2, sem_ia2, sem_ib2,
               acc):
    c = lax.axis_index("c")
    s = lax.axis_index("s")
    wid = c * NS + s
    r0 = s * RPT
    base = wid * CPT
    pltpu.sync_copy(zeros.at[pl.ds(r0, RPT)], acc.at[pl.ds(r0, RPT)])
    pltpu.sync_copy(packed.at[pl.ds(base, 1)], ia)
    pltpu.sync_copy(packed.at[pl.ds(base + 1, 1)], ib)
    plsc.subcore_barrier()

    pltpu.async_copy(h.at[ia.at[0, 0]], rows_a, sem_ga)
    pltpu.async_copy(packed.at[pl.ds(base + 2, 1)], ia2, sem_ia2)

    def body(i, carry):
        k = 4 * i
        pltpu.async_copy(h.at[ib.at[0, 0]], rows_b, sem_gb)
        pltpu.async_copy(packed.at[pl.ds(base + k + 3, 1)], ib2, sem_ib2)
        pltpu.make_async_copy(h.at[ia.at[0, 0]], rows_a, sem_ga).wait()
        pltpu.sync_copy(rows_a, acc.at[ia.at[0, 1]], add=True)
        pltpu.make_async_copy(packed.at[pl.ds(base, 1)], ia2, sem_ia2).wait()
        pltpu.async_copy(h.at[ia2.at[0, 0]], rows_a, sem_ga)
        pltpu.async_copy(packed.at[pl.ds(base + k + 4, 1)], ia, sem_ia)
        pltpu.make_async_copy(h.at[ib.at[0, 0]], rows_b, sem_gb).wait()
        pltpu.sync_copy(rows_b, acc.at[ib.at[0, 1]], add=True)
        pltpu.make_async_copy(packed.at[pl.ds(base, 1)], ib2, sem_ib2).wait()
        pltpu.async_copy(h.at[ib2.at[0, 0]], rows_b, sem_gb)
        pltpu.async_copy(packed.at[pl.ds(base + k + 5, 1)], ib, sem_ib)
        pltpu.make_async_copy(h.at[ia2.at[0, 0]], rows_a, sem_ga).wait()
        pltpu.sync_copy(rows_a, acc.at[ia2.at[0, 1]], add=True)
        pltpu.make_async_copy(packed.at[pl.ds(base, 1)], ia, sem_ia).wait()
        pltpu.async_copy(h.at[ia.at[0, 0]], rows_a, sem_ga)
        pltpu.async_copy(packed.at[pl.ds(base + k + 6, 1)], ia2, sem_ia2)
        pltpu.make_async_copy(h.at[ib2.at[0, 0]], rows_b, sem_gb).wait()
        pltpu.sync_copy(rows_b, acc.at[ib2.at[0, 1]], add=True)
        pltpu.make_async_copy(packed.at[pl.ds(base, 1)], ib, sem_ib).wait()
        return carry

    lax.fori_loop(0, CPT // 4 - 1, body, 0)
    pltpu.async_copy(h.at[ib.at[0, 0]], rows_b, sem_gb)
    pltpu.make_async_copy(h.at[ia.at[0, 0]], rows_a, sem_ga).wait()
    pltpu.sync_copy(rows_a, acc.at[ia.at[0, 1]], add=True)
    pltpu.make_async_copy(packed.at[pl.ds(base, 1)], ia2, sem_ia2).wait()
    pltpu.async_copy(h.at[ia2.at[0, 0]], rows_a, sem_ga)
    pltpu.make_async_copy(h.at[ib.at[0, 0]], rows_b, sem_gb).wait()
    pltpu.sync_copy(rows_b, acc.at[ib.at[0, 1]], add=True)
    pltpu.sync_copy(packed.at[pl.ds(base + CPT - 1, 1)], ib2)
    pltpu.async_copy(h.at[ib2.at[0, 0]], rows_b, sem_gb)
    pltpu.make_async_copy(h.at[ia2.at[0, 0]], rows_a, sem_ga).wait()
    pltpu.sync_copy(rows_a, acc.at[ia2.at[0, 1]], add=True)
    pltpu.make_async_copy(h.at[ib2.at[0, 0]], rows_b, sem_gb).wait()
    pltpu.sync_copy(rows_b, acc.at[ib2.at[0, 1]], add=True)

    plsc.subcore_barrier()
    pltpu.sync_copy(acc.at[pl.ds(r0, RPT)], out.at[c, pl.ds(r0, RPT)])


_edge_call = pl.kernel(
    _edge_body,
    out_type=jax.ShapeDtypeStruct((NC, NPAD, D), jnp.float32),
    mesh=_mesh,
    scratch_types=[
        pltpu.VMEM((1, 2, CHUNK), jnp.int32),
        pltpu.VMEM((1, 2, CHUNK), jnp.int32),
        pltpu.VMEM((CHUNK, D), jnp.float32),
        pltpu.VMEM((CHUNK, D), jnp.float32),
        pltpu.SemaphoreType.DMA,
        pltpu.SemaphoreType.DMA,
        pltpu.SemaphoreType.DMA,
        pltpu.SemaphoreType.DMA,
        pltpu.VMEM((1, 2, CHUNK), jnp.int32),
        pltpu.VMEM((1, 2, CHUNK), jnp.int32),
        pltpu.SemaphoreType.DMA,
        pltpu.SemaphoreType.DMA,
        pltpu.VMEM_SHARED((NPAD, D), jnp.float32),
    ],
)


def _tc0_body(degp_ref, dis2_ref):
    dsum = degp_ref[0]
    for i in range(1, NW):
        dsum = dsum + degp_ref[i]
    dis2_ref[...] = lax.rsqrt(dsum + 1.0)


_tc0_call = pl.pallas_call(
    _tc0_body,
    out_shape=jax.ShapeDtypeStruct((HR, D), jnp.float32),
)


def _tc1_body(x_ref, w_ref, dis_ref, hp_ref):
    h = jnp.dot(x_ref[...], w_ref[...], preferred_element_type=jnp.float32)
    hp_ref[...] = h * dis_ref[...]


_tc1_call = pl.pallas_call(
    _tc1_body,
    out_shape=jax.ShapeDtypeStruct((N, D), jnp.float32),
)


def _tc2_body(s_ref, hp_ref, dis_ref, b_ref, w_ref, out_ref):
    dis = dis_ref[...]
    t = (s_ref[0, :N] + s_ref[1, :N] + hp_ref[...]) * dis + b_ref[...]
    h1 = jnp.maximum(t, 0.0)
    out_ref[...] = jnp.dot(h1, w_ref[...], preferred_element_type=jnp.float32) * dis


_tc2_call = pl.pallas_call(
    _tc2_body,
    out_shape=jax.ShapeDtypeStruct((N, D), jnp.float32),
)


def _tc3_body(s_ref, hp_ref, dis_ref, b_ref, out_ref):
    t = (s_ref[0, :N] + s_ref[1, :N] + hp_ref[...]) * dis_ref[...] + b_ref[...]
    out_ref[...] = 1.0 / (1.0 + jnp.exp(-t))


_tc3_call = pl.pallas_call(
    _tc3_body,
    out_shape=jax.ShapeDtypeStruct((N, D), jnp.float32),
)


@jax.jit
def kernel(x, edge_index, W0, b0, W1, b1):
    src = edge_index[0]
    dst = edge_index[1]
    pad = E_PAD - E
    iota_pad = jnp.arange(pad, dtype=jnp.int32)
    src_pad = (iota_pad * 131) % N
    dst_pad = N + iota_pad % (NPAD - N)
    src2 = jnp.concatenate([src, src_pad]).reshape(-1, CHUNK)
    dst2 = jnp.concatenate([dst, dst_pad]).reshape(-1, CHUNK)
    packed = jnp.stack([src2, dst2], axis=1)

    zeros = jnp.zeros((NPAD, D), jnp.float32)

    degp = _deg_call(dst2)
    dis = _tc0_call(degp).reshape(NPAD, 1)[:N]
    h0p = _tc1_call(x, W0, dis)

    s1 = _edge_call(h0p, packed, zeros)
    h1p = _tc2_call(s1, h0p, dis, b0.reshape(1, D), W1)

    s2 = _edge_call(h1p, packed, zeros)
    return _tc3_call(s2, h1p, dis, b1.reshape(1, D))
